# baseline (speedup 1.0000x reference)
import jax
import jax.numpy as jnp
from jax.experimental import pallas as pl
from jax.experimental.pallas import tpu as pltpu

_N_CHUNKS = 2


def _make_body(b, n, d, n_chunks):
    rows = n // n_chunks

    def body(pe_hbm, out_hbm, vmem, in_sems, out_sems):
        for c in range(n_chunks):
            sl = pl.ds(c * rows, rows)
            pltpu.make_async_copy(pe_hbm.at[sl], vmem.at[sl], in_sems.at[c]).start()
        for c in range(n_chunks):
            sl = pl.ds(c * rows, rows)
            pltpu.make_async_copy(pe_hbm.at[sl], vmem.at[sl], in_sems.at[c]).wait()
            for i in range(b):
                pltpu.make_async_copy(
                    vmem.at[sl], out_hbm.at[i, sl], out_sems.at[c, i]
                ).start()
        for c in range(n_chunks):
            sl = pl.ds(c * rows, rows)
            for i in range(b):
                pltpu.make_async_copy(
                    vmem.at[sl], out_hbm.at[i, sl], out_sems.at[c, i]
                ).wait()

    return body


def kernel(x, pe_weight):
    b = x.shape[0]
    n, d = pe_weight.shape
    n_chunks = _N_CHUNKS if n % _N_CHUNKS == 0 else 1
    return pl.pallas_call(
        _make_body(b, n, d, n_chunks),
        in_specs=[pl.BlockSpec(memory_space=pl.ANY)],
        out_specs=pl.BlockSpec(memory_space=pl.ANY),
        out_shape=jax.ShapeDtypeStruct((b, n, d), pe_weight.dtype),
        scratch_shapes=[
            pltpu.VMEM((n, d), pe_weight.dtype),
            pltpu.SemaphoreType.DMA((n_chunks,)),
            pltpu.SemaphoreType.DMA((n_chunks, b)),
        ],
    )(pe_weight)


# speedup vs baseline: 1.0605x; 1.0605x over previous
import jax
import jax.numpy as jnp
from jax.experimental import pallas as pl
from jax.experimental.pallas import tpu as pltpu

_N_CHUNKS = 4


def _make_body(b, n, d, n_chunks):
    rows = n // n_chunks

    def body(pe_hbm, out_hbm, vmem, in_sems, out_sems):
        for c in range(n_chunks):
            sl = pl.ds(c * rows, rows)
            pltpu.make_async_copy(pe_hbm.at[sl], vmem.at[sl], in_sems.at[c]).start()
        for c in range(n_chunks):
            sl = pl.ds(c * rows, rows)
            pltpu.make_async_copy(pe_hbm.at[sl], vmem.at[sl], in_sems.at[c]).wait()
            for i in range(b):
                pltpu.make_async_copy(
                    vmem.at[sl], out_hbm.at[i, sl], out_sems.at[c, i]
                ).start()
        for c in range(n_chunks):
            sl = pl.ds(c * rows, rows)
            for i in range(b):
                pltpu.make_async_copy(
                    vmem.at[sl], out_hbm.at[i, sl], out_sems.at[c, i]
                ).wait()

    return body


def kernel(x, pe_weight):
    b = x.shape[0]
    n, d = pe_weight.shape
    n_chunks = _N_CHUNKS if n % _N_CHUNKS == 0 else 1
    return pl.pallas_call(
        _make_body(b, n, d, n_chunks),
        in_specs=[pl.BlockSpec(memory_space=pl.ANY)],
        out_specs=pl.BlockSpec(memory_space=pl.ANY),
        out_shape=jax.ShapeDtypeStruct((b, n, d), pe_weight.dtype),
        scratch_shapes=[
            pltpu.VMEM((n, d), pe_weight.dtype),
            pltpu.SemaphoreType.DMA((n_chunks,)),
            pltpu.SemaphoreType.DMA((n_chunks, b)),
        ],
    )(pe_weight)
